# baseline (speedup 1.0000x reference)
import jax
import jax.numpy as jnp
from jax import lax
from jax.experimental import pallas as pl
from jax.experimental.pallas import tpu as pltpu
from jax.experimental.pallas import tpu_sc as plsc

N = 1024
DH = 256
DE = 16
BI = 128
NB_TC = 6
N_TC = NB_TC * BI
N_SC = N - N_TC
NW = 32
ROWS_W = N_SC // NW

_BF = jnp.bfloat16


def _dot(p, q):
    return jnp.dot(p.astype(_BF), q.astype(_BF),
                   preferred_element_type=jnp.float32)


def _dot32(p, q):
    return jnp.dot(p, q, preferred_element_type=jnp.float32)


def _gru_layer(adjm, x, emsg, wmsg_ref, wz_ref, uz_ref, wr_ref, ur_ref,
               wh_ref, uh_ref, bz_ref, br_ref, bh_ref):
    m = _dot(adjm, _dot(x, wmsg_ref[...])) * (1.0 / N) + emsg
    z = jax.nn.sigmoid(_dot(m, wz_ref[...]) + _dot(x, uz_ref[...])
                       + bz_ref[...])
    r = jax.nn.sigmoid(_dot(m, wr_ref[...]) + _dot(x, ur_ref[...])
                       + br_ref[...])
    hh = jnp.tanh(_dot(m, wh_ref[...]) + _dot(r * x, uh_ref[...])
                  + bh_ref[...])
    return (1.0 - z) * x + z * hh


def _sc_eagg_body(et_hbm, adj_hbm, out_hbm, adj_v, ebuf0, ebuf1, res_v,
                  sem0, sem1):
    wid = lax.axis_index("s") * 2 + lax.axis_index("c")
    base = N_TC + wid * ROWS_W
    pltpu.sync_copy(adj_hbm.at[pl.ds(base, ROWS_W)], adj_v)
    pltpu.make_async_copy(et_hbm.at[base], ebuf0, sem0).start()
    bufs = (ebuf0, ebuf1)
    sems = (sem0, sem1)
    for r in range(ROWS_W):
        buf, sem = bufs[r % 2], sems[r % 2]
        pltpu.make_async_copy(et_hbm.at[base + r], buf, sem).wait()
        if r + 1 < ROWS_W:
            pltpu.make_async_copy(et_hbm.at[base + r + 1],
                                  bufs[(r + 1) % 2], sems[(r + 1) % 2]).start()

        def jb_body(jb, accs):
            av = adj_v[r, pl.ds(jb * 16, 16)]
            return tuple(accs[c] + buf[c, pl.ds(jb * 16, 16)] * av
                         for c in range(DE))

        accs = lax.fori_loop(
            0, N // 16, jb_body,
            tuple(jnp.zeros((16,), jnp.float32) for _ in range(DE)))
        for c in range(DE):
            res_v[r, pl.ds(c * 16, 16)] = accs[c]
    pltpu.sync_copy(res_v, out_hbm.at[pl.ds(wid * ROWS_W, ROWS_W)])


def _tc_stream_body(adj_ref, et_ref, we_ref, h_ref, wmsg_ref, wz_ref, uz_ref,
                    wr_ref, ur_ref, wh_ref, uh_ref, bz_ref, br_ref, bh_ref,
                    emsg_ref, x1_ref, xw_ref):
    i = pl.program_id(0)
    inv_n = 1.0 / N
    rows = pl.ds(i * BI, BI)

    @pl.when(i == 0)
    def _():
        xw_ref[...] = _dot(h_ref[...], wmsg_ref[...])

    a = adj_ref[rows, :]
    eagg = jnp.sum(et_ref[...] * a[:, None, :], axis=2)
    em = _dot32(eagg, we_ref[...]) * inv_n
    emsg_ref[...] = em

    hb = h_ref[rows, :]
    m1 = _dot(a, xw_ref[...]) * inv_n + em
    z1 = jax.nn.sigmoid(_dot(m1, wz_ref[...]) + _dot(hb, uz_ref[...])
                        + bz_ref[...])
    r1 = jax.nn.sigmoid(_dot(m1, wr_ref[...]) + _dot(hb, ur_ref[...])
                        + br_ref[...])
    hh1 = jnp.tanh(_dot(m1, wh_ref[...]) + _dot(r1 * hb, uh_ref[...])
                   + bh_ref[...])
    x1_ref[...] = (1.0 - z1) * hb + z1 * hh1


def _tc_tail_body(adj_ref, h_ref, eaggsc_ref, emsgtc_ref, x1tc_ref, f2_ref,
                  wmsg_ref, wz_ref, uz_ref, wr_ref, ur_ref, wh_ref, uh_ref,
                  bz_ref, br_ref, bh_ref, wg_ref, bg_ref, wo_ref, bo_ref,
                  node_ref, wnemb_ref, wfc_ref, bfc_ref, out_ref):
    inv_n = 1.0 / N
    em_sc = _dot32(eaggsc_ref[...], f2_ref[...]) * inv_n
    emsg = jnp.concatenate([emsgtc_ref[...], em_sc], axis=0)

    a_sc = adj_ref[N_TC:, :]
    hb = h_ref[N_TC:, :]
    xw = _dot(h_ref[...], wmsg_ref[...])
    m1 = _dot(a_sc, xw) * inv_n + em_sc
    z1 = jax.nn.sigmoid(_dot(m1, wz_ref[...]) + _dot(hb, uz_ref[...])
                        + bz_ref[...])
    r1 = jax.nn.sigmoid(_dot(m1, wr_ref[...]) + _dot(hb, ur_ref[...])
                        + br_ref[...])
    hh1 = jnp.tanh(_dot(m1, wh_ref[...]) + _dot(r1 * hb, uh_ref[...])
                   + bh_ref[...])
    x1_sc = (1.0 - z1) * hb + z1 * hh1

    x = jnp.concatenate([x1tc_ref[...], x1_sc], axis=0)
    adjm = adj_ref[...].astype(_BF)
    for _ in range(2):
        x = _gru_layer(adjm, x, emsg, wmsg_ref, wz_ref, uz_ref, wr_ref,
                       ur_ref, wh_ref, uh_ref, bz_ref, br_ref, bh_ref)
    gate = jax.nn.sigmoid(_dot(x, wg_ref[...]) + bg_ref[...])
    hv = gate * jnp.tanh(_dot(x, wo_ref[...]) + bo_ref[...])
    gv = jnp.sum(hv, axis=0, keepdims=True)
    ne = _dot32(node_ref[...], wnemb_ref[...])
    out_ref[...] = (_dot32(gv, wfc_ref[:DH, :]) + _dot32(ne, wfc_ref[DH:, :])
                    + bfc_ref[...])


def kernel(h, e, adj, node, W_msg, W_e, Wz, Uz, Wr, Ur, Wh, Uh, bz, br, bh,
           W_g, b_g, W_o, b_o, W_nemb, W_fc, b_fc):
    adj2 = adj.reshape(N, N)
    et = jnp.transpose(e.reshape(N, N, DE), (0, 2, 1))
    h2 = h.reshape(N, DH)
    bz2, br2, bh2 = (x.reshape(1, DH) for x in (bz, br, bh))
    f2 = jnp.repeat(W_e, 16, axis=0)

    eagg_sc = pl.kernel(
        _sc_eagg_body,
        out_type=jax.ShapeDtypeStruct((N_SC, DE * 16), jnp.float32),
        mesh=plsc.VectorSubcoreMesh(core_axis_name="c", subcore_axis_name="s"),
        scratch_types=[
            pltpu.VMEM((ROWS_W, N), jnp.float32),
            pltpu.VMEM((DE, N), jnp.float32),
            pltpu.VMEM((DE, N), jnp.float32),
            pltpu.VMEM((ROWS_W, DE * 16), jnp.float32),
            pltpu.SemaphoreType.DMA,
            pltpu.SemaphoreType.DMA,
        ],
    )(et, adj2)

    full = lambda *shape: pl.BlockSpec(shape, lambda i: (0,) * len(shape))
    emsg_tc, x1_tc, _ = pl.pallas_call(
        _tc_stream_body,
        grid=(NB_TC,),
        in_specs=[
            full(N, N),
            pl.BlockSpec((BI, DE, N), lambda i: (i, 0, 0)),
            full(DE, DH),
            full(N, DH),
            full(DH, DH), full(DH, DH), full(DH, DH),
            full(DH, DH), full(DH, DH), full(DH, DH),
            full(DH, DH),
            full(1, DH), full(1, DH), full(1, DH),
        ],
        out_specs=[
            pl.BlockSpec((BI, DH), lambda i: (i, 0)),
            pl.BlockSpec((BI, DH), lambda i: (i, 0)),
            full(N, DH),
        ],
        out_shape=[
            jax.ShapeDtypeStruct((N_TC, DH), jnp.float32),
            jax.ShapeDtypeStruct((N_TC, DH), jnp.float32),
            jax.ShapeDtypeStruct((N, DH), jnp.float32),
        ],
    )(adj2, et, W_e, h2, W_msg, Wz, Uz, Wr, Ur, Wh, Uh, bz2, br2, bh2)

    out = pl.pallas_call(
        _tc_tail_body,
        out_shape=jax.ShapeDtypeStruct((1, DH), jnp.float32),
    )(adj2, h2, eagg_sc, emsg_tc, x1_tc, f2, W_msg, Wz, Uz, Wr, Ur, Wh, Uh,
      bz2, br2, bh2, W_g, b_g.reshape(1, DH), W_o, b_o.reshape(1, DH),
      node.reshape(1, 128), W_nemb, W_fc, b_fc.reshape(1, DH))

    return out.reshape(DH)


# speedup vs baseline: 1.6399x; 1.6399x over previous
import jax
import jax.numpy as jnp
from jax.experimental import pallas as pl
from jax.experimental.pallas import tpu as pltpu

N = 1024
DH = 256
DE = 16
BI = 128
NB = N // BI

_BF = jnp.bfloat16


def _dot(p, q):
    return jnp.dot(p.astype(_BF), q.astype(_BF),
                   preferred_element_type=jnp.float32)


def _dot32(p, q):
    return jnp.dot(p, q, preferred_element_type=jnp.float32)


def _sig(a):
    return 0.5 + 0.5 * jnp.tanh(0.5 * a)


def _fused_body(adj_ref, et_ref, we_ref, h_ref, wmsg_ref, wz_ref, uz_ref,
                wr_ref, ur_ref, wh_ref, uh_ref, bz_ref, br_ref, bh_ref,
                wg_ref, bg_ref, wo_ref, bo_ref, node_ref, wnemb_ref, wfc_ref,
                bfc_ref, out_ref, emsg_ref, xw_ref, x1_ref, m2a_ref):
    i = pl.program_id(0)
    inv_n = 1.0 / N
    rows = pl.ds(i * BI, BI)

    @pl.when(i == 0)
    def _():
        xw_ref[...] = _dot(h_ref[...], wmsg_ref[...])

    a = adj_ref[rows, :]
    eagg = jnp.sum(et_ref[...] * a[:, None, :], axis=2)
    em = _dot32(eagg, we_ref[...]) * inv_n
    emsg_ref[rows, :] = em

    hb = h_ref[rows, :]
    m1 = _dot(a, xw_ref[...]) * inv_n + em
    z1 = _sig(_dot(m1, wz_ref[...]) + _dot(hb, uz_ref[...])
                        + bz_ref[...])
    r1 = _sig(_dot(m1, wr_ref[...]) + _dot(hb, ur_ref[...])
                        + br_ref[...])
    hh1 = jnp.tanh(_dot(m1, wh_ref[...]) + _dot(r1 * hb, uh_ref[...])
                   + bh_ref[...])
    x1 = (1.0 - z1) * hb + z1 * hh1
    x1_ref[rows, :] = x1

    @pl.when(i == 0)
    def _():
        m2a_ref[...] = jnp.zeros((N, DH), jnp.float32)
    m2a_ref[...] += _dot(adj_ref[:, rows].astype(_BF), _dot(x1, wmsg_ref[...]))

    @pl.when(i == NB - 1)
    def _():
        adjm = adj_ref[...].astype(_BF)
        emsg = emsg_ref[...]
        x = x1_ref[...]
        m = m2a_ref[...] * inv_n + emsg
        for layer in range(2):
            z = _sig(_dot(m, wz_ref[...]) + _dot(x, uz_ref[...])
                               + bz_ref[...])
            r = _sig(_dot(m, wr_ref[...]) + _dot(x, ur_ref[...])
                               + br_ref[...])
            hh = jnp.tanh(_dot(m, wh_ref[...]) + _dot(r * x, uh_ref[...])
                          + bh_ref[...])
            x = (1.0 - z) * x + z * hh
            if layer == 0:
                m = _dot(adjm, _dot(x, wmsg_ref[...])) * inv_n + emsg
        gate = _sig(_dot(x, wg_ref[...]) + bg_ref[...])
        hv = gate * jnp.tanh(_dot(x, wo_ref[...]) + bo_ref[...])
        gv = jnp.sum(hv, axis=0, keepdims=True)
        ne = _dot32(node_ref[...], wnemb_ref[...])
        out_ref[...] = (_dot32(gv, wfc_ref[:DH, :]) + _dot32(ne, wfc_ref[DH:, :])
                        + bfc_ref[...])


def kernel(h, e, adj, node, W_msg, W_e, Wz, Uz, Wr, Ur, Wh, Uh, bz, br, bh,
           W_g, b_g, W_o, b_o, W_nemb, W_fc, b_fc):
    adj2 = adj.reshape(N, N)
    et = jnp.transpose(e.reshape(N, N, DE), (0, 2, 1))
    h2 = h.reshape(N, DH)

    full = lambda *shape: pl.BlockSpec(shape, lambda i: (0,) * len(shape))
    out = pl.pallas_call(
        _fused_body,
        grid=(NB,),
        in_specs=[
            full(N, N),
            pl.BlockSpec((BI, DE, N), lambda i: (i, 0, 0)),
            full(DE, DH),
            full(N, DH),
            full(DH, DH), full(DH, DH), full(DH, DH),
            full(DH, DH), full(DH, DH), full(DH, DH),
            full(DH, DH),
            full(1, DH), full(1, DH), full(1, DH),
            full(DH, DH), full(1, DH),
            full(DH, DH), full(1, DH),
            full(1, 128), full(128, DH),
            full(2 * DH, DH), full(1, DH),
        ],
        out_specs=full(1, DH),
        out_shape=jax.ShapeDtypeStruct((1, DH), jnp.float32),
        scratch_shapes=[pltpu.VMEM((N, DH), jnp.float32),
                        pltpu.VMEM((N, DH), jnp.float32),
                        pltpu.VMEM((N, DH), jnp.float32),
                        pltpu.VMEM((N, DH), jnp.float32)],
    )(adj2, et, W_e, h2, W_msg, Wz, Uz, Wr, Ur, Wh, Uh,
      bz.reshape(1, DH), br.reshape(1, DH), bh.reshape(1, DH),
      W_g, b_g.reshape(1, DH), W_o, b_o.reshape(1, DH),
      node.reshape(1, 128), W_nemb, W_fc, b_fc.reshape(1, DH))

    return out.reshape(DH)


# speedup vs baseline: 1.6705x; 1.0187x over previous
import jax
import jax.numpy as jnp
from jax.experimental import pallas as pl
from jax.experimental.pallas import tpu as pltpu

N = 1024
DH = 256
DE = 16
BI = 128
NB = N // BI

_BF = jnp.bfloat16


def _dot(p, q):
    return jnp.dot(p.astype(_BF), q.astype(_BF),
                   preferred_element_type=jnp.float32)


def _dot32(p, q):
    return jnp.dot(p, q, preferred_element_type=jnp.float32)


def _sig(a):
    return 0.5 + 0.5 * jnp.tanh(0.5 * a)


def _fused_body(adj_ref, et_ref, we_ref, h_ref, wmsg_ref, wz_ref, uz_ref,
                wr_ref, ur_ref, wh_ref, uh_ref, bz_ref, br_ref, bh_ref,
                wg_ref, bg_ref, wo_ref, bo_ref, node_ref, wnemb_ref, wfc_ref,
                bfc_ref, out_ref, emsg_ref, xw_ref, x1_ref):
    i = pl.program_id(0)
    inv_n = 1.0 / N
    rows = pl.ds(i * BI, BI)

    @pl.when(i == 0)
    def _():
        xw_ref[...] = _dot(h_ref[...], wmsg_ref[...])

    a = adj_ref[rows, :]
    eagg = jnp.sum(et_ref[...] * a[:, None, :], axis=2)
    em = _dot32(eagg, we_ref[...]) * inv_n
    emsg_ref[rows, :] = em

    hb = h_ref[rows, :]
    m1 = _dot(a, xw_ref[...]) * inv_n + em
    z1 = _sig(_dot(m1, wz_ref[...]) + _dot(hb, uz_ref[...])
                        + bz_ref[...])
    r1 = _sig(_dot(m1, wr_ref[...]) + _dot(hb, ur_ref[...])
                        + br_ref[...])
    hh1 = jnp.tanh(_dot(m1, wh_ref[...]) + _dot(r1 * hb, uh_ref[...])
                   + bh_ref[...])
    x1_ref[rows, :] = (1.0 - z1) * hb + z1 * hh1

    @pl.when(i == NB - 1)
    def _():
        adjm = adj_ref[...].astype(_BF)
        emsg = emsg_ref[...]
        x = x1_ref[...]
        for _ in range(2):
            m = _dot(adjm, _dot(x, wmsg_ref[...])) * inv_n + emsg
            z = _sig(_dot(m, wz_ref[...]) + _dot(x, uz_ref[...])
                               + bz_ref[...])
            r = _sig(_dot(m, wr_ref[...]) + _dot(x, ur_ref[...])
                               + br_ref[...])
            hh = jnp.tanh(_dot(m, wh_ref[...]) + _dot(r * x, uh_ref[...])
                          + bh_ref[...])
            x = (1.0 - z) * x + z * hh
        gate = _sig(_dot(x, wg_ref[...]) + bg_ref[...])
        hv = gate * jnp.tanh(_dot(x, wo_ref[...]) + bo_ref[...])
        gv = jnp.sum(hv, axis=0, keepdims=True)
        ne = _dot32(node_ref[...], wnemb_ref[...])
        out_ref[...] = (_dot32(gv, wfc_ref[:DH, :]) + _dot32(ne, wfc_ref[DH:, :])
                        + bfc_ref[...])


def kernel(h, e, adj, node, W_msg, W_e, Wz, Uz, Wr, Ur, Wh, Uh, bz, br, bh,
           W_g, b_g, W_o, b_o, W_nemb, W_fc, b_fc):
    adj2 = adj.reshape(N, N)
    et = jnp.transpose(e.reshape(N, N, DE), (0, 2, 1))
    h2 = h.reshape(N, DH)

    full = lambda *shape: pl.BlockSpec(shape, lambda i: (0,) * len(shape))
    out = pl.pallas_call(
        _fused_body,
        grid=(NB,),
        in_specs=[
            full(N, N),
            pl.BlockSpec((BI, DE, N), lambda i: (i, 0, 0)),
            full(DE, DH),
            full(N, DH),
            full(DH, DH), full(DH, DH), full(DH, DH),
            full(DH, DH), full(DH, DH), full(DH, DH),
            full(DH, DH),
            full(1, DH), full(1, DH), full(1, DH),
            full(DH, DH), full(1, DH),
            full(DH, DH), full(1, DH),
            full(1, 128), full(128, DH),
            full(2 * DH, DH), full(1, DH),
        ],
        out_specs=full(1, DH),
        out_shape=jax.ShapeDtypeStruct((1, DH), jnp.float32),
        scratch_shapes=[pltpu.VMEM((N, DH), jnp.float32),
                        pltpu.VMEM((N, DH), jnp.float32),
                        pltpu.VMEM((N, DH), jnp.float32)],
    )(adj2, et, W_e, h2, W_msg, Wz, Uz, Wr, Ur, Wh, Uh,
      bz.reshape(1, DH), br.reshape(1, DH), bh.reshape(1, DH),
      W_g, b_g.reshape(1, DH), W_o, b_o.reshape(1, DH),
      node.reshape(1, 128), W_nemb, W_fc, b_fc.reshape(1, DH))

    return out.reshape(DH)
